# baseline (speedup 1.0000x reference)
import jax
import jax.numpy as jnp
from jax.experimental import pallas as pl
from jax.experimental.pallas import tpu as pltpu

_BATCH = 8
_D_IN = 64
_VOCAB = 1_000_000
_BLOCK_V = 65536


def _matmul_body(state_ref, w_ref, out_ref):
    out_ref[...] = jnp.dot(
        state_ref[...], w_ref[...], preferred_element_type=jnp.float32
    )


def kernel(state, W):
    grid = pl.cdiv(_VOCAB, _BLOCK_V)
    return pl.pallas_call(
        _matmul_body,
        grid=(grid,),
        in_specs=[
            pl.BlockSpec((_BATCH, _D_IN), lambda i: (0, 0)),
            pl.BlockSpec((_D_IN, _BLOCK_V), lambda i: (0, i)),
        ],
        out_specs=pl.BlockSpec((_BATCH, _BLOCK_V), lambda i: (0, i)),
        out_shape=jax.ShapeDtypeStruct((_BATCH, _VOCAB), jnp.float32),
        compiler_params=pltpu.CompilerParams(
            dimension_semantics=("parallel",),
        ),
    )(state, W)


# speedup vs baseline: 1.0034x; 1.0034x over previous
import jax
import jax.numpy as jnp
from jax.experimental import pallas as pl
from jax.experimental.pallas import tpu as pltpu

_BATCH = 8
_D_IN = 64
_VOCAB = 1_000_000
_BLOCK_V = 40960


def _matmul_body(state_ref, w_ref, out_ref):
    out_ref[...] = jnp.dot(
        state_ref[...], w_ref[...], preferred_element_type=jnp.float32
    )


def kernel(state, W):
    grid = pl.cdiv(_VOCAB, _BLOCK_V)
    return pl.pallas_call(
        _matmul_body,
        grid=(grid,),
        in_specs=[
            pl.BlockSpec((_BATCH, _D_IN), lambda i: (0, 0)),
            pl.BlockSpec((_D_IN, _BLOCK_V), lambda i: (0, i)),
        ],
        out_specs=pl.BlockSpec((_BATCH, _BLOCK_V), lambda i: (0, i)),
        out_shape=jax.ShapeDtypeStruct((_BATCH, _VOCAB), jnp.float32),
        compiler_params=pltpu.CompilerParams(
            dimension_semantics=("parallel",),
        ),
    )(state, W)
